# baseline (speedup 1.0000x reference)
import functools

import jax
import jax.numpy as jnp
import numpy as np
from jax import lax
from jax.experimental import pallas as pl
from jax.experimental.pallas import tpu as pltpu
from jax.experimental.pallas import tpu_sc as plsc

_B_TOTAL = 4096 * 200
_NUM_ROWS = 100
_EMB_D = 32
_LANES = 16
_NT = 7
_ROWS_PAD = _NT * _LANES
_UNROLL = 8

_B_PER_W = _B_TOTAL // 32
_B_STREAM = 21248
_B_CHAIN = _B_PER_W - _B_STREAM
_B_S1 = 10624
_B_S2 = _B_STREAM - _B_S1

_TIDX = np.zeros((_EMB_D * _ROWS_PAD,), np.int32)
for _c in range(_EMB_D):
    for _r in range(_ROWS_PAD):
        _TIDX[_c * _ROWS_PAD + _r] = (_r % _NUM_ROWS) * _EMB_D + _c

_GATHER_DNUMS = lax.GatherDimensionNumbers(
    offset_dims=(), collapsed_slice_dims=(0,), start_index_map=(0,))


def _dyng(tab, sub):
    return lax.gather(tab, sub[:, None], dimension_numbers=_GATHER_DNUMS,
                      slice_sizes=(1,),
                      mode=lax.GatherScatterMode.PROMISE_IN_BOUNDS)


@jax.jit
def _sc_mean_lookup(qry_flat, emb_flat, tidx):
    info = plsc.get_sparse_core_info()
    NC, NS, L = info.num_cores, info.num_subcores, info.num_lanes
    NW = NC * NS
    b_per_w = _B_TOTAL // NW
    mesh = plsc.VectorSubcoreMesh(core_axis_name="c", subcore_axis_name="s")

    @functools.partial(
        pl.kernel,
        mesh=mesh,
        out_type=jax.ShapeDtypeStruct((_B_TOTAL,), jnp.float32),
        scratch_types=[
            pltpu.VMEM((_B_STREAM,), jnp.int32),
            pltpu.VMEM((_B_CHAIN,), jnp.int32),
            pltpu.VMEM((b_per_w,), jnp.float32),
            pltpu.VMEM((_EMB_D * _ROWS_PAD,), jnp.int32),
            pltpu.VMEM((_EMB_D * _ROWS_PAD,), jnp.float32),
            pltpu.VMEM((_ROWS_PAD,), jnp.float32),
            pltpu.VMEM_SHARED((_ROWS_PAD,), jnp.float32),
            pltpu.SemaphoreType.DMA,
            pltpu.SemaphoreType.DMA,
            pltpu.SemaphoreType.DMA,
            pltpu.SemaphoreType.DMA,
        ],
    )
    def k(qry_hbm, emb_hbm, tidx_hbm, out_hbm,
          idxs_v, idxc_v, out_v, tidx_v, embT_v, means_v, means_sh,
          sem_idx, sem_idxc, sem_t, sem_g):
        wid = lax.axis_index("s") * NC + lax.axis_index("c")
        sid = lax.axis_index("s")
        base = wid * b_per_w

        idxs_dma = pltpu.async_copy(qry_hbm.at[pl.ds(base, _B_STREAM)],
                                    idxs_v, sem_idx)
        idxc_dma = pltpu.async_copy(
            qry_hbm.at[pl.ds(base + _B_STREAM, _B_CHAIN)], idxc_v, sem_idxc)

        @pl.when(sid == 0)
        def _():
            pltpu.sync_copy(tidx_hbm, tidx_v)
            pltpu.async_copy(emb_hbm.at[tidx_v], embT_v, sem_t).wait()
            for b in range(_NT):
                acc = embT_v[pl.ds(b * L, L)]
                for c in range(1, _EMB_D):
                    acc = acc + embT_v[pl.ds(c * _ROWS_PAD + b * L, L)]
                means_v[pl.ds(b * L, L)] = acc * (1.0 / _EMB_D)
            pltpu.sync_copy(means_v, means_sh)

        plsc.subcore_barrier()
        idxs_dma.wait()

        g1 = pltpu.async_copy(means_sh.at[idxs_v.at[pl.ds(0, _B_S1)]],
                              out_v.at[pl.ds(0, _B_S1)], sem_g)
        g2 = pltpu.async_copy(means_sh.at[idxs_v.at[pl.ds(_B_S1, _B_S2)]],
                              out_v.at[pl.ds(_B_S1, _B_S2)], sem_t)

        pltpu.sync_copy(means_sh, means_v)
        tabs = [means_v[pl.ds(t * L, L)] for t in range(_NT)]
        idxc_dma.wait()

        def step(ii):
            idx = idxc_v[pl.ds(ii, L)]
            sub = idx & (L - 1)
            res = _dyng(tabs[0], sub)
            for t in range(1, _NT):
                res = jnp.where(idx >= t * L, _dyng(tabs[t], sub), res)
            out_v[pl.ds(_B_STREAM + ii, L)] = res

        def body(i, carry):
            ii = i * (L * _UNROLL)
            for u in range(_UNROLL):
                step(ii + u * L)
            return carry

        lax.fori_loop(0, _B_CHAIN // (L * _UNROLL), body, jnp.int32(0))

        g1.wait()
        o1 = pltpu.async_copy(out_v.at[pl.ds(0, _B_S1)],
                              out_hbm.at[pl.ds(base, _B_S1)], sem_idx)
        g2.wait()
        pltpu.sync_copy(out_v.at[pl.ds(_B_S1, b_per_w - _B_S1)],
                        out_hbm.at[pl.ds(base + _B_S1, b_per_w - _B_S1)])
        o1.wait()

    return k(qry_flat, emb_flat, tidx)


def kernel(q_seq, r_seq, qry_seq, emb_table):
    B, S = qry_seq.shape
    qry_flat = qry_seq.reshape(-1).astype(jnp.int32)
    emb_flat = emb_table.reshape(-1)
    out = _sc_mean_lookup(qry_flat, emb_flat, jnp.asarray(_TIDX))
    return out.reshape(B, S)


# speedup vs baseline: 1.0446x; 1.0446x over previous
import functools

import jax
import jax.numpy as jnp
import numpy as np
from jax import lax
from jax.experimental import pallas as pl
from jax.experimental.pallas import tpu as pltpu
from jax.experimental.pallas import tpu_sc as plsc

_B_TOTAL = 4096 * 200
_NUM_ROWS = 100
_EMB_D = 32
_LANES = 16
_NT = 7
_ROWS_PAD = _NT * _LANES
_UNROLL = 8

_B_PER_W = _B_TOTAL // 32
_B_STREAM = 17920
_B_CHAIN = _B_PER_W - _B_STREAM

_TIDX = np.zeros((_EMB_D * _ROWS_PAD,), np.int32)
for _c in range(_EMB_D):
    for _r in range(_ROWS_PAD):
        _TIDX[_c * _ROWS_PAD + _r] = (_r % _NUM_ROWS) * _EMB_D + _c

_GATHER_DNUMS = lax.GatherDimensionNumbers(
    offset_dims=(), collapsed_slice_dims=(0,), start_index_map=(0,))


def _dyng(tab, sub):
    return lax.gather(tab, sub[:, None], dimension_numbers=_GATHER_DNUMS,
                      slice_sizes=(1,),
                      mode=lax.GatherScatterMode.PROMISE_IN_BOUNDS)


@jax.jit
def _sc_mean_lookup(qry_flat, emb_flat, tidx):
    info = plsc.get_sparse_core_info()
    NC, NS, L = info.num_cores, info.num_subcores, info.num_lanes
    NW = NC * NS
    b_per_w = _B_TOTAL // NW
    mesh = plsc.VectorSubcoreMesh(core_axis_name="c", subcore_axis_name="s")

    @functools.partial(
        pl.kernel,
        mesh=mesh,
        out_type=jax.ShapeDtypeStruct((_B_TOTAL,), jnp.float32),
        scratch_types=[
            pltpu.VMEM((_B_STREAM,), jnp.int32),
            pltpu.VMEM((_B_CHAIN,), jnp.int32),
            pltpu.VMEM((b_per_w,), jnp.float32),
            pltpu.VMEM((_EMB_D * _ROWS_PAD,), jnp.int32),
            pltpu.VMEM((_EMB_D * _ROWS_PAD,), jnp.float32),
            pltpu.VMEM((_ROWS_PAD,), jnp.float32),
            pltpu.VMEM_SHARED((_ROWS_PAD,), jnp.float32),
            pltpu.SemaphoreType.DMA,
            pltpu.SemaphoreType.DMA,
            pltpu.SemaphoreType.DMA,
            pltpu.SemaphoreType.DMA,
        ],
    )
    def k(qry_hbm, emb_hbm, tidx_hbm, out_hbm,
          idxs_v, idxc_v, out_v, tidx_v, embT_v, means_v, means_sh,
          sem_idx, sem_idxc, sem_t, sem_g):
        wid = lax.axis_index("s") * NC + lax.axis_index("c")
        sid = lax.axis_index("s")
        base = wid * b_per_w

        idxs_dma = pltpu.async_copy(qry_hbm.at[pl.ds(base, _B_STREAM)],
                                    idxs_v, sem_idx)
        idxc_dma = pltpu.async_copy(
            qry_hbm.at[pl.ds(base + _B_STREAM, _B_CHAIN)], idxc_v, sem_idxc)

        @pl.when(sid == 0)
        def _():
            pltpu.sync_copy(tidx_hbm, tidx_v)
            pltpu.async_copy(emb_hbm.at[tidx_v], embT_v, sem_t).wait()
            for b in range(_NT):
                acc = embT_v[pl.ds(b * L, L)]
                for c in range(1, _EMB_D):
                    acc = acc + embT_v[pl.ds(c * _ROWS_PAD + b * L, L)]
                means_v[pl.ds(b * L, L)] = acc * (1.0 / _EMB_D)
            pltpu.sync_copy(means_v, means_sh)

        plsc.subcore_barrier()
        idxs_dma.wait()

        gather_dma = pltpu.async_copy(means_sh.at[idxs_v],
                                      out_v.at[pl.ds(0, _B_STREAM)], sem_g)

        pltpu.sync_copy(means_sh, means_v)
        tabs = [means_v[pl.ds(t * L, L)] for t in range(_NT)]
        idxc_dma.wait()

        def step(ii):
            idx = idxc_v[pl.ds(ii, L)]
            sub = idx & (L - 1)
            res = _dyng(tabs[0], sub)
            for t in range(1, _NT):
                res = jnp.where(idx >= t * L, _dyng(tabs[t], sub), res)
            out_v[pl.ds(_B_STREAM + ii, L)] = res

        def body(i, carry):
            ii = i * (L * _UNROLL)
            for u in range(_UNROLL):
                step(ii + u * L)
            return carry

        lax.fori_loop(0, _B_CHAIN // (L * _UNROLL), body, jnp.int32(0))

        gather_dma.wait()
        pltpu.sync_copy(out_v, out_hbm.at[pl.ds(base, b_per_w)])

    return k(qry_flat, emb_flat, tidx)


def kernel(q_seq, r_seq, qry_seq, emb_table):
    B, S = qry_seq.shape
    qry_flat = qry_seq.reshape(-1).astype(jnp.int32)
    emb_flat = emb_table.reshape(-1)
    out = _sc_mean_lookup(qry_flat, emb_flat, jnp.asarray(_TIDX))
    return out.reshape(B, S)


# speedup vs baseline: 1.0577x; 1.0125x over previous
import functools

import jax
import jax.numpy as jnp
import numpy as np
from jax import lax
from jax.experimental import pallas as pl
from jax.experimental.pallas import tpu as pltpu
from jax.experimental.pallas import tpu_sc as plsc

_B_TOTAL = 4096 * 200
_NUM_ROWS = 100
_EMB_D = 32
_LANES = 16
_NT = 7
_ROWS_PAD = _NT * _LANES
_UNROLL = 8

_B_PER_W = _B_TOTAL // 32
_B_STREAM = 16128
_B_CHAIN = _B_PER_W - _B_STREAM

_TIDX = np.zeros((_EMB_D * _ROWS_PAD,), np.int32)
for _c in range(_EMB_D):
    for _r in range(_ROWS_PAD):
        _TIDX[_c * _ROWS_PAD + _r] = (_r % _NUM_ROWS) * _EMB_D + _c

_GATHER_DNUMS = lax.GatherDimensionNumbers(
    offset_dims=(), collapsed_slice_dims=(0,), start_index_map=(0,))


def _dyng(tab, sub):
    return lax.gather(tab, sub[:, None], dimension_numbers=_GATHER_DNUMS,
                      slice_sizes=(1,),
                      mode=lax.GatherScatterMode.PROMISE_IN_BOUNDS)


@jax.jit
def _sc_mean_lookup(qry_flat, emb_flat, tidx):
    info = plsc.get_sparse_core_info()
    NC, NS, L = info.num_cores, info.num_subcores, info.num_lanes
    NW = NC * NS
    b_per_w = _B_TOTAL // NW
    mesh = plsc.VectorSubcoreMesh(core_axis_name="c", subcore_axis_name="s")

    @functools.partial(
        pl.kernel,
        mesh=mesh,
        out_type=jax.ShapeDtypeStruct((_B_TOTAL,), jnp.float32),
        scratch_types=[
            pltpu.VMEM((_B_STREAM,), jnp.int32),
            pltpu.VMEM((_B_CHAIN,), jnp.int32),
            pltpu.VMEM((b_per_w,), jnp.float32),
            pltpu.VMEM((_EMB_D * _ROWS_PAD,), jnp.int32),
            pltpu.VMEM((_EMB_D * _ROWS_PAD,), jnp.float32),
            pltpu.VMEM((_ROWS_PAD,), jnp.float32),
            pltpu.VMEM_SHARED((_ROWS_PAD,), jnp.float32),
            pltpu.SemaphoreType.DMA,
            pltpu.SemaphoreType.DMA,
            pltpu.SemaphoreType.DMA,
            pltpu.SemaphoreType.DMA,
        ],
    )
    def k(qry_hbm, emb_hbm, tidx_hbm, out_hbm,
          idxs_v, idxc_v, out_v, tidx_v, embT_v, means_v, means_sh,
          sem_idx, sem_idxc, sem_t, sem_g):
        wid = lax.axis_index("s") * NC + lax.axis_index("c")
        sid = lax.axis_index("s")
        base = wid * b_per_w

        idxs_dma = pltpu.async_copy(qry_hbm.at[pl.ds(base, _B_STREAM)],
                                    idxs_v, sem_idx)
        idxc_dma = pltpu.async_copy(
            qry_hbm.at[pl.ds(base + _B_STREAM, _B_CHAIN)], idxc_v, sem_idxc)

        @pl.when(sid == 0)
        def _():
            pltpu.sync_copy(tidx_hbm, tidx_v)
            pltpu.async_copy(emb_hbm.at[tidx_v], embT_v, sem_t).wait()
            for b in range(_NT):
                acc = embT_v[pl.ds(b * L, L)]
                for c in range(1, _EMB_D):
                    acc = acc + embT_v[pl.ds(c * _ROWS_PAD + b * L, L)]
                means_v[pl.ds(b * L, L)] = acc * (1.0 / _EMB_D)
            pltpu.sync_copy(means_v, means_sh)

        plsc.subcore_barrier()
        idxs_dma.wait()

        gather_dma = pltpu.async_copy(means_sh.at[idxs_v],
                                      out_v.at[pl.ds(0, _B_STREAM)], sem_g)

        pltpu.sync_copy(means_sh, means_v)
        tabs = [means_v[pl.ds(t * L, L)] for t in range(_NT)]
        idxc_dma.wait()

        def step(ii):
            idx = idxc_v[pl.ds(ii, L)]
            sub = idx & (L - 1)
            res = _dyng(tabs[0], sub)
            for t in range(1, _NT):
                res = jnp.where(idx >= t * L, _dyng(tabs[t], sub), res)
            out_v[pl.ds(_B_STREAM + ii, L)] = res

        def body(i, carry):
            ii = i * (L * _UNROLL)
            for u in range(_UNROLL):
                step(ii + u * L)
            return carry

        lax.fori_loop(0, _B_CHAIN // (L * _UNROLL), body, jnp.int32(0))

        gather_dma.wait()
        pltpu.sync_copy(out_v, out_hbm.at[pl.ds(base, b_per_w)])

    return k(qry_flat, emb_flat, tidx)


def kernel(q_seq, r_seq, qry_seq, emb_table):
    B, S = qry_seq.shape
    qry_flat = qry_seq.reshape(-1).astype(jnp.int32)
    emb_flat = emb_table.reshape(-1)
    out = _sc_mean_lookup(qry_flat, emb_flat, jnp.asarray(_TIDX))
    return out.reshape(B, S)


# speedup vs baseline: 1.0719x; 1.0135x over previous
import functools

import jax
import jax.numpy as jnp
import numpy as np
from jax import lax
from jax.experimental import pallas as pl
from jax.experimental.pallas import tpu as pltpu
from jax.experimental.pallas import tpu_sc as plsc

_B_TOTAL = 4096 * 200
_NUM_ROWS = 100
_EMB_D = 32
_LANES = 16
_NT = 7
_ROWS_PAD = _NT * _LANES
_UNROLL = 8

_B_PER_W = _B_TOTAL // 32
_B_STREAM = 14336
_B_CHAIN = _B_PER_W - _B_STREAM

_TIDX = np.zeros((_EMB_D * _ROWS_PAD,), np.int32)
for _c in range(_EMB_D):
    for _r in range(_ROWS_PAD):
        _TIDX[_c * _ROWS_PAD + _r] = (_r % _NUM_ROWS) * _EMB_D + _c

_GATHER_DNUMS = lax.GatherDimensionNumbers(
    offset_dims=(), collapsed_slice_dims=(0,), start_index_map=(0,))


def _dyng(tab, sub):
    return lax.gather(tab, sub[:, None], dimension_numbers=_GATHER_DNUMS,
                      slice_sizes=(1,),
                      mode=lax.GatherScatterMode.PROMISE_IN_BOUNDS)


@jax.jit
def _sc_mean_lookup(qry_flat, emb_flat, tidx):
    info = plsc.get_sparse_core_info()
    NC, NS, L = info.num_cores, info.num_subcores, info.num_lanes
    NW = NC * NS
    b_per_w = _B_TOTAL // NW
    mesh = plsc.VectorSubcoreMesh(core_axis_name="c", subcore_axis_name="s")

    @functools.partial(
        pl.kernel,
        mesh=mesh,
        out_type=jax.ShapeDtypeStruct((_B_TOTAL,), jnp.float32),
        scratch_types=[
            pltpu.VMEM((_B_STREAM,), jnp.int32),
            pltpu.VMEM((_B_CHAIN,), jnp.int32),
            pltpu.VMEM((b_per_w,), jnp.float32),
            pltpu.VMEM((_EMB_D * _ROWS_PAD,), jnp.int32),
            pltpu.VMEM((_EMB_D * _ROWS_PAD,), jnp.float32),
            pltpu.VMEM((_ROWS_PAD,), jnp.float32),
            pltpu.VMEM_SHARED((_ROWS_PAD,), jnp.float32),
            pltpu.SemaphoreType.DMA,
            pltpu.SemaphoreType.DMA,
            pltpu.SemaphoreType.DMA,
            pltpu.SemaphoreType.DMA,
        ],
    )
    def k(qry_hbm, emb_hbm, tidx_hbm, out_hbm,
          idxs_v, idxc_v, out_v, tidx_v, embT_v, means_v, means_sh,
          sem_idx, sem_idxc, sem_t, sem_g):
        wid = lax.axis_index("s") * NC + lax.axis_index("c")
        sid = lax.axis_index("s")
        base = wid * b_per_w

        idxs_dma = pltpu.async_copy(qry_hbm.at[pl.ds(base, _B_STREAM)],
                                    idxs_v, sem_idx)
        idxc_dma = pltpu.async_copy(
            qry_hbm.at[pl.ds(base + _B_STREAM, _B_CHAIN)], idxc_v, sem_idxc)

        @pl.when(sid == 0)
        def _():
            pltpu.sync_copy(tidx_hbm, tidx_v)
            pltpu.async_copy(emb_hbm.at[tidx_v], embT_v, sem_t).wait()
            for b in range(_NT):
                acc = embT_v[pl.ds(b * L, L)]
                for c in range(1, _EMB_D):
                    acc = acc + embT_v[pl.ds(c * _ROWS_PAD + b * L, L)]
                means_v[pl.ds(b * L, L)] = acc * (1.0 / _EMB_D)
            pltpu.sync_copy(means_v, means_sh)

        plsc.subcore_barrier()
        idxs_dma.wait()

        gather_dma = pltpu.async_copy(means_sh.at[idxs_v],
                                      out_v.at[pl.ds(0, _B_STREAM)], sem_g)

        pltpu.sync_copy(means_sh, means_v)
        tabs = [means_v[pl.ds(t * L, L)] for t in range(_NT)]
        idxc_dma.wait()

        def step(ii):
            idx = idxc_v[pl.ds(ii, L)]
            sub = idx & (L - 1)
            res = _dyng(tabs[0], sub)
            for t in range(1, _NT):
                res = jnp.where(idx >= t * L, _dyng(tabs[t], sub), res)
            out_v[pl.ds(_B_STREAM + ii, L)] = res

        def body(i, carry):
            ii = i * (L * _UNROLL)
            for u in range(_UNROLL):
                step(ii + u * L)
            return carry

        lax.fori_loop(0, _B_CHAIN // (L * _UNROLL), body, jnp.int32(0))

        gather_dma.wait()
        pltpu.sync_copy(out_v, out_hbm.at[pl.ds(base, b_per_w)])

    return k(qry_flat, emb_flat, tidx)


def kernel(q_seq, r_seq, qry_seq, emb_table):
    B, S = qry_seq.shape
    qry_flat = qry_seq.reshape(-1).astype(jnp.int32)
    emb_flat = emb_table.reshape(-1)
    out = _sc_mean_lookup(qry_flat, emb_flat, jnp.asarray(_TIDX))
    return out.reshape(B, S)


# speedup vs baseline: 1.0951x; 1.0216x over previous
import functools

import jax
import jax.numpy as jnp
import numpy as np
from jax import lax
from jax.experimental import pallas as pl
from jax.experimental.pallas import tpu as pltpu
from jax.experimental.pallas import tpu_sc as plsc

_B_TOTAL = 4096 * 200
_NUM_ROWS = 100
_EMB_D = 32
_LANES = 16
_NT = 7
_ROWS_PAD = _NT * _LANES
_UNROLL = 8

_B_PER_W = _B_TOTAL // 32
_B_STREAM = 11264
_B_CHAIN = _B_PER_W - _B_STREAM

_TIDX = np.zeros((_EMB_D * _ROWS_PAD,), np.int32)
for _c in range(_EMB_D):
    for _r in range(_ROWS_PAD):
        _TIDX[_c * _ROWS_PAD + _r] = (_r % _NUM_ROWS) * _EMB_D + _c

_GATHER_DNUMS = lax.GatherDimensionNumbers(
    offset_dims=(), collapsed_slice_dims=(0,), start_index_map=(0,))


def _dyng(tab, sub):
    return lax.gather(tab, sub[:, None], dimension_numbers=_GATHER_DNUMS,
                      slice_sizes=(1,),
                      mode=lax.GatherScatterMode.PROMISE_IN_BOUNDS)


@jax.jit
def _sc_mean_lookup(qry_flat, emb_flat, tidx):
    info = plsc.get_sparse_core_info()
    NC, NS, L = info.num_cores, info.num_subcores, info.num_lanes
    NW = NC * NS
    b_per_w = _B_TOTAL // NW
    mesh = plsc.VectorSubcoreMesh(core_axis_name="c", subcore_axis_name="s")

    @functools.partial(
        pl.kernel,
        mesh=mesh,
        out_type=jax.ShapeDtypeStruct((_B_TOTAL,), jnp.float32),
        scratch_types=[
            pltpu.VMEM((_B_STREAM,), jnp.int32),
            pltpu.VMEM((_B_CHAIN,), jnp.int32),
            pltpu.VMEM((b_per_w,), jnp.float32),
            pltpu.VMEM((_EMB_D * _ROWS_PAD,), jnp.int32),
            pltpu.VMEM((_EMB_D * _ROWS_PAD,), jnp.float32),
            pltpu.VMEM((_ROWS_PAD,), jnp.float32),
            pltpu.VMEM_SHARED((_ROWS_PAD,), jnp.float32),
            pltpu.SemaphoreType.DMA,
            pltpu.SemaphoreType.DMA,
            pltpu.SemaphoreType.DMA,
            pltpu.SemaphoreType.DMA,
        ],
    )
    def k(qry_hbm, emb_hbm, tidx_hbm, out_hbm,
          idxs_v, idxc_v, out_v, tidx_v, embT_v, means_v, means_sh,
          sem_idx, sem_idxc, sem_t, sem_g):
        wid = lax.axis_index("s") * NC + lax.axis_index("c")
        sid = lax.axis_index("s")
        base = wid * b_per_w

        idxs_dma = pltpu.async_copy(qry_hbm.at[pl.ds(base, _B_STREAM)],
                                    idxs_v, sem_idx)
        idxc_dma = pltpu.async_copy(
            qry_hbm.at[pl.ds(base + _B_STREAM, _B_CHAIN)], idxc_v, sem_idxc)

        @pl.when(sid == 0)
        def _():
            pltpu.sync_copy(tidx_hbm, tidx_v)
            pltpu.async_copy(emb_hbm.at[tidx_v], embT_v, sem_t).wait()
            for b in range(_NT):
                acc = embT_v[pl.ds(b * L, L)]
                for c in range(1, _EMB_D):
                    acc = acc + embT_v[pl.ds(c * _ROWS_PAD + b * L, L)]
                means_v[pl.ds(b * L, L)] = acc * (1.0 / _EMB_D)
            pltpu.sync_copy(means_v, means_sh)

        plsc.subcore_barrier()
        idxs_dma.wait()

        gather_dma = pltpu.async_copy(means_sh.at[idxs_v],
                                      out_v.at[pl.ds(0, _B_STREAM)], sem_g)

        pltpu.sync_copy(means_sh, means_v)
        tabs = [means_v[pl.ds(t * L, L)] for t in range(_NT)]
        idxc_dma.wait()

        def step(ii):
            idx = idxc_v[pl.ds(ii, L)]
            sub = idx & (L - 1)
            res = _dyng(tabs[0], sub)
            for t in range(1, _NT):
                res = jnp.where(idx >= t * L, _dyng(tabs[t], sub), res)
            out_v[pl.ds(_B_STREAM + ii, L)] = res

        def body(i, carry):
            ii = i * (L * _UNROLL)
            for u in range(_UNROLL):
                step(ii + u * L)
            return carry

        lax.fori_loop(0, _B_CHAIN // (L * _UNROLL), body, jnp.int32(0))

        gather_dma.wait()
        pltpu.sync_copy(out_v, out_hbm.at[pl.ds(base, b_per_w)])

    return k(qry_flat, emb_flat, tidx)


def kernel(q_seq, r_seq, qry_seq, emb_table):
    B, S = qry_seq.shape
    qry_flat = qry_seq.reshape(-1).astype(jnp.int32)
    emb_flat = emb_table.reshape(-1)
    out = _sc_mean_lookup(qry_flat, emb_flat, jnp.asarray(_TIDX))
    return out.reshape(B, S)
